# baseline (speedup 1.0000x reference)
import functools

import jax
import jax.numpy as jnp
from jax import lax
from jax.experimental import pallas as pl
from jax.experimental.pallas import tpu as pltpu
from jax.experimental.pallas import tpu_sc as plsc

B, L, D = 4096, 50, 32
NC, NS = 2, 16
NW = NC * NS
SPW = B // NW
DT, DL = D // 8, 8
BT, BL = B // 128, 128


def _make_emb_kernel(interpret=False):
    mesh = plsc.VectorSubcoreMesh(core_axis_name="c", subcore_axis_name="s",
                                  num_cores=NC, num_subcores=NS)

    @functools.partial(
        pl.kernel,
        out_type=jax.ShapeDtypeStruct((L, DT, BT, DL, BL), jnp.float32),
        mesh=mesh,
        scratch_types=[
            pltpu.VMEM((L, SPW), jnp.int32),
            pltpu.VMEM((2, SPW, D), jnp.float32),
            pltpu.VMEM((2, DT, DL, BL), jnp.float32),
            pltpu.SemaphoreType.DMA((2,)),
            pltpu.SemaphoreType.DMA((2,)),
        ],
        compiler_params=pltpu.CompilerParams(use_tc_tiling_on_sc=False,
                                             needs_layout_passes=False),
        interpret=interpret,
    )
    def emb_kernel(tokT, tab, out, idx_v, bufA, bufT, gsem, wsem):
        cc = lax.axis_index("c")
        ss = lax.axis_index("s")
        wid = ss * NC + cc

        pltpu.sync_copy(tokT.at[:, pl.ds(wid * SPW, SPW)], idx_v)

        iota16 = lax.iota(jnp.int32, 16)

        def g_start(l, b):
            pltpu.async_copy(tab.at[idx_v.at[l]], bufA.at[b], gsem.at[b])

        def g_wait(b):
            pltpu.make_async_copy(tab.at[pl.ds(0, SPW)], bufA.at[b],
                                  gsem.at[b]).wait()

        def w_start(l, b):
            pltpu.async_copy(bufT.at[b], out.at[l, :, wid], wsem.at[b])

        def w_wait(l, b):
            pltpu.make_async_copy(bufT.at[b], out.at[l, :, wid],
                                  wsem.at[b]).wait()

        def transpose(b):
            @pl.loop(0, 8)
            def _(c):
                rows = iota16 + 16 * c
                for d in range(D):
                    v = plsc.load_gather(
                        bufA.at[b], [rows, jnp.full((16,), d, jnp.int32)])
                    bufT[b, d // DL, d % DL, pl.ds(16 * c, 16)] = v

        for b in range(2):
            g_start(b, b)
        for b in range(2):
            g_wait(b)
            transpose(b)
            w_start(b, b)
            g_start(b + 2, b)

        @pl.loop(2, L - 2, step=2)
        def _(l0):
            for b in range(2):
                l = l0 + b
                g_wait(b)
                w_wait(l - 2, b)
                transpose(b)
                w_start(l, b)
                g_start(l + 2, b)

        for b in range(2):
            l = L - 2 + b
            g_wait(b)
            w_wait(l - 2, b)
            transpose(b)
            w_start(l, b)
        for b in range(2):
            w_wait(L - 2 + b, b)

    return emb_kernel


_emb = _make_emb_kernel()


def kernel(src_tokens, tgt_tokens, src_table, tgt_table):
    outs = []
    for tok, tab in ((src_tokens, src_table), (tgt_tokens, tgt_table)):
        y = _emb(jnp.transpose(tok).astype(jnp.int32), tab)
        outs.append(y.transpose(2, 4, 0, 1, 3).reshape(B, L, D))
    return tuple(outs)


# speedup vs baseline: 1.1837x; 1.1837x over previous
import functools

import jax
import jax.numpy as jnp
from jax import lax
from jax.experimental import pallas as pl
from jax.experimental.pallas import tpu as pltpu
from jax.experimental.pallas import tpu_sc as plsc

B, L, D = 4096, 50, 32
NC, NS = 2, 16
NW = NC * NS
SPW = B // NW
DT, DL = D // 8, 8
BT, BL = B // 128, 128


def _make_emb_kernel(interpret=False):
    mesh = plsc.VectorSubcoreMesh(core_axis_name="c", subcore_axis_name="s",
                                  num_cores=NC, num_subcores=NS)

    @functools.partial(
        pl.kernel,
        out_type=jax.ShapeDtypeStruct((L, DT, BT, DL, BL), jnp.float32),
        mesh=mesh,
        scratch_types=[
            pltpu.VMEM((L, SPW), jnp.int32),
            pltpu.VMEM((2, SPW, D), jnp.float32),
            pltpu.VMEM((2, DT, DL, BL), jnp.float32),
            pltpu.SemaphoreType.DMA((2,)),
            pltpu.SemaphoreType.DMA((2,)),
        ],
        compiler_params=pltpu.CompilerParams(use_tc_tiling_on_sc=False,
                                             needs_layout_passes=False),
        interpret=interpret,
    )
    def emb_kernel(tokT, tab, out, idx_v, bufA, bufT, gsem, wsem):
        cc = lax.axis_index("c")
        ss = lax.axis_index("s")
        wid = ss * NC + cc

        pltpu.sync_copy(tokT.at[:, pl.ds(wid * SPW, SPW)], idx_v)

        iota16 = lax.iota(jnp.int32, 16)
        zero16 = jnp.zeros((16,), jnp.int32)
        dl_v = jnp.bitwise_and(iota16, 7)
        dt_lo = lax.shift_right_logical(iota16, 3)
        dt_hi = dt_lo + 2

        def g_start(l, b):
            pltpu.async_copy(tab.at[idx_v.at[l]], bufA.at[b], gsem.at[b])

        def g_wait(b):
            pltpu.make_async_copy(tab.at[pl.ds(0, SPW)], bufA.at[b],
                                  gsem.at[b]).wait()

        def w_start(l, b):
            pltpu.async_copy(bufT.at[b], out.at[l, :, wid], wsem.at[b])

        def w_wait(l, b):
            pltpu.make_async_copy(bufT.at[b], out.at[l, :, wid],
                                  wsem.at[b]).wait()

        def transpose(b):
            @pl.loop(0, SPW, unroll=8)
            def _(k):
                kv = zero16 + k
                v0 = bufA[b, k, pl.ds(0, 16)]
                plsc.store_scatter(bufT.at[b], [dt_lo, dl_v, kv], v0)
                v1 = bufA[b, k, pl.ds(16, 16)]
                plsc.store_scatter(bufT.at[b], [dt_hi, dl_v, kv], v1)

        for b in range(2):
            g_start(b, b)
        for b in range(2):
            g_wait(b)
            transpose(b)
            w_start(b, b)
            g_start(b + 2, b)

        @pl.loop(2, L - 2, step=2)
        def _(l0):
            for b in range(2):
                l = l0 + b
                g_wait(b)
                w_wait(l - 2, b)
                transpose(b)
                w_start(l, b)
                g_start(l + 2, b)

        for b in range(2):
            l = L - 2 + b
            g_wait(b)
            w_wait(l - 2, b)
            transpose(b)
            w_start(l, b)
        for b in range(2):
            w_wait(L - 2 + b, b)

    return emb_kernel


_emb = _make_emb_kernel()


def kernel(src_tokens, tgt_tokens, src_table, tgt_table):
    outs = []
    for tok, tab in ((src_tokens, src_table), (tgt_tokens, tgt_table)):
        y = _emb(jnp.transpose(tok).astype(jnp.int32), tab)
        outs.append(y.transpose(2, 4, 0, 1, 3).reshape(B, L, D))
    return tuple(outs)


# speedup vs baseline: 1.3681x; 1.1557x over previous
import functools

import jax
import jax.numpy as jnp
from jax import lax
from jax.experimental import pallas as pl
from jax.experimental.pallas import tpu as pltpu
from jax.experimental.pallas import tpu_sc as plsc

B, L, D = 4096, 50, 32
NC, NS = 2, 16
NW = NC * NS
SPW = B // NW
DT, DL = D // 8, 8
BT, BL = B // 128, 128
NB = 8


def _make_transposed_kernel(interpret=False):
    mesh = plsc.VectorSubcoreMesh(core_axis_name="c", subcore_axis_name="s",
                                  num_cores=NC, num_subcores=NS)

    @functools.partial(
        pl.kernel,
        out_type=jax.ShapeDtypeStruct((L, DT, BT, DL, BL), jnp.float32),
        mesh=mesh,
        scratch_types=[
            pltpu.VMEM((L, SPW), jnp.int32),
            pltpu.VMEM((2, SPW, D), jnp.float32),
            pltpu.VMEM((2, DT, DL, BL), jnp.float32),
            pltpu.SemaphoreType.DMA((2,)),
            pltpu.SemaphoreType.DMA((2,)),
        ],
        compiler_params=pltpu.CompilerParams(use_tc_tiling_on_sc=False,
                                             needs_layout_passes=False),
        interpret=interpret,
    )
    def emb_kernel(tokT, tab, out, idx_v, bufA, bufT, gsem, wsem):
        cc = lax.axis_index("c")
        ss = lax.axis_index("s")
        wid = ss * NC + cc

        pltpu.sync_copy(tokT.at[:, pl.ds(wid * SPW, SPW)], idx_v)

        iota16 = lax.iota(jnp.int32, 16)
        zero16 = jnp.zeros((16,), jnp.int32)
        dl_v = jnp.bitwise_and(iota16, 7)
        dt_lo = lax.shift_right_logical(iota16, 3)
        dt_hi = dt_lo + 2

        def g_start(l, b):
            pltpu.async_copy(tab.at[idx_v.at[l]], bufA.at[b], gsem.at[b])

        def g_wait(b):
            pltpu.make_async_copy(tab.at[pl.ds(0, SPW)], bufA.at[b],
                                  gsem.at[b]).wait()

        def w_start(l, b):
            pltpu.async_copy(bufT.at[b], out.at[l, :, wid], wsem.at[b])

        def w_wait(l, b):
            pltpu.make_async_copy(bufT.at[b], out.at[l, :, wid],
                                  wsem.at[b]).wait()

        def transpose(b):
            @pl.loop(0, SPW, unroll=8)
            def _(k):
                kv = zero16 + k
                v0 = bufA[b, k, pl.ds(0, 16)]
                plsc.store_scatter(bufT.at[b], [dt_lo, dl_v, kv], v0)
                v1 = bufA[b, k, pl.ds(16, 16)]
                plsc.store_scatter(bufT.at[b], [dt_hi, dl_v, kv], v1)

        for b in range(2):
            g_start(b, b)
        for b in range(2):
            g_wait(b)
            transpose(b)
            w_start(b, b)
            g_start(b + 2, b)

        @pl.loop(2, L - 2, step=2)
        def _(l0):
            for b in range(2):
                l = l0 + b
                g_wait(b)
                w_wait(l - 2, b)
                transpose(b)
                w_start(l, b)
                g_start(l + 2, b)

        for b in range(2):
            l = L - 2 + b
            g_wait(b)
            w_wait(l - 2, b)
            transpose(b)
            w_start(l, b)
        for b in range(2):
            w_wait(L - 2 + b, b)

    return emb_kernel


def _make_bulk_kernel(interpret=False):
    mesh = plsc.VectorSubcoreMesh(core_axis_name="c", subcore_axis_name="s",
                                  num_cores=NC, num_subcores=NS)

    @functools.partial(
        pl.kernel,
        out_type=jax.ShapeDtypeStruct((B, L, D), jnp.float32),
        mesh=mesh,
        scratch_types=[
            pltpu.VMEM((SPW, L), jnp.int32),
            pltpu.VMEM((NB, L, D), jnp.float32),
            pltpu.SemaphoreType.DMA((NB,)),
            pltpu.SemaphoreType.DMA((NB,)),
        ],
        compiler_params=pltpu.CompilerParams(use_tc_tiling_on_sc=False),
        interpret=interpret,
    )
    def emb_kernel(tok, tab, out, idx_v, ring, gsem, wsem):
        cc = lax.axis_index("c")
        ss = lax.axis_index("s")
        wid = ss * NC + cc
        base = wid * SPW

        pltpu.sync_copy(tok.at[pl.ds(base, SPW)], idx_v)

        def g_start(k, b):
            pltpu.async_copy(tab.at[idx_v.at[k]], ring.at[b], gsem.at[b])

        def g_wait(b):
            pltpu.make_async_copy(tab.at[pl.ds(0, L)], ring.at[b],
                                  gsem.at[b]).wait()

        def w_start(k, b):
            pltpu.async_copy(ring.at[b], out.at[base + k], wsem.at[b])

        def w_wait(k, b):
            pltpu.make_async_copy(ring.at[b], out.at[base + k],
                                  wsem.at[b]).wait()

        for b in range(NB):
            g_start(b, b)

        @pl.loop(0, SPW - NB, step=NB)
        def _(k0):
            for b in range(NB):
                k = k0 + b
                g_wait(b)
                w_start(k, b)
                w_wait(k, b)
                g_start(k + NB, b)

        for b in range(NB):
            g_wait(b)
            w_start(SPW - NB + b, b)
        for b in range(NB):
            w_wait(SPW - NB + b, b)

    return emb_kernel


_emb_t = _make_transposed_kernel()
_emb_bulk = _make_bulk_kernel()


def kernel(src_tokens, tgt_tokens, src_table, tgt_table):
    tgt_emb = _emb_bulk(tgt_tokens.astype(jnp.int32), tgt_table)
    y = _emb_t(jnp.transpose(src_tokens).astype(jnp.int32), src_table)
    src_emb = y.transpose(2, 4, 0, 1, 3).reshape(B, L, D)
    return src_emb, tgt_emb


# speedup vs baseline: 1.3746x; 1.0048x over previous
import functools

import jax
import jax.numpy as jnp
from jax import lax
from jax.experimental import pallas as pl
from jax.experimental.pallas import tpu as pltpu
from jax.experimental.pallas import tpu_sc as plsc

B, L, D = 4096, 50, 32
NC, NS = 2, 16
NW = NC * NS
SPW = B // NW
DT, DL = D // 8, 8
BT, BL = B // 128, 128
NB = 8


def _make_transposed_kernel(interpret=False):
    mesh = plsc.VectorSubcoreMesh(core_axis_name="c", subcore_axis_name="s",
                                  num_cores=NC, num_subcores=NS)

    @functools.partial(
        pl.kernel,
        out_type=jax.ShapeDtypeStruct((L, DT, BT * DL * BL), jnp.float32),
        mesh=mesh,
        scratch_types=[
            pltpu.VMEM((L, SPW), jnp.int32),
            pltpu.VMEM((2, SPW, D), jnp.float32),
            pltpu.VMEM((2, DT, DL * BL), jnp.float32),
            pltpu.SemaphoreType.DMA((2,)),
            pltpu.SemaphoreType.DMA((2,)),
        ],
        compiler_params=pltpu.CompilerParams(use_tc_tiling_on_sc=False,
                                             needs_layout_passes=False),
        interpret=interpret,
    )
    def emb_kernel(tokT, tab, out, idx_v, bufA, bufT, gsem, wsem):
        cc = lax.axis_index("c")
        ss = lax.axis_index("s")
        wid = ss * NC + cc

        pltpu.sync_copy(tokT.at[:, pl.ds(wid * SPW, SPW)], idx_v)

        iota16 = lax.iota(jnp.int32, 16)
        zero16 = jnp.zeros((16,), jnp.int32)
        dl128 = jnp.bitwise_and(iota16, 7) * 128
        dt_lo = lax.shift_right_logical(iota16, 3)
        dt_hi = dt_lo + 2

        def g_start(l, b):
            pltpu.async_copy(tab.at[idx_v.at[l]], bufA.at[b], gsem.at[b])

        def g_wait(b):
            pltpu.make_async_copy(tab.at[pl.ds(0, SPW)], bufA.at[b],
                                  gsem.at[b]).wait()

        def w_start(l, b):
            pltpu.async_copy(bufT.at[b],
                             out.at[l, :, pl.ds(wid * DL * BL, DL * BL)],
                             wsem.at[b])

        def w_wait(l, b):
            pltpu.make_async_copy(bufT.at[b],
                                  out.at[l, :, pl.ds(wid * DL * BL, DL * BL)],
                                  wsem.at[b]).wait()

        def transpose(b):
            @pl.loop(0, SPW, unroll=8)
            def _(k):
                inner = dl128 + (zero16 + k)
                v0 = bufA[b, k, pl.ds(0, 16)]
                plsc.store_scatter(bufT.at[b], [dt_lo, inner], v0)
                v1 = bufA[b, k, pl.ds(16, 16)]
                plsc.store_scatter(bufT.at[b], [dt_hi, inner], v1)

        for b in range(2):
            g_start(b, b)
        for b in range(2):
            g_wait(b)
            transpose(b)
            w_start(b, b)
            g_start(b + 2, b)

        @pl.loop(2, L - 2, step=2)
        def _(l0):
            for b in range(2):
                l = l0 + b
                g_wait(b)
                w_wait(l - 2, b)
                transpose(b)
                w_start(l, b)
                g_start(l + 2, b)

        for b in range(2):
            l = L - 2 + b
            g_wait(b)
            w_wait(l - 2, b)
            transpose(b)
            w_start(l, b)
        for b in range(2):
            w_wait(L - 2 + b, b)

    return emb_kernel


def _make_bulk_kernel(interpret=False):
    mesh = plsc.VectorSubcoreMesh(core_axis_name="c", subcore_axis_name="s",
                                  num_cores=NC, num_subcores=NS)

    @functools.partial(
        pl.kernel,
        out_type=jax.ShapeDtypeStruct((B, L, D), jnp.float32),
        mesh=mesh,
        scratch_types=[
            pltpu.VMEM((SPW, L), jnp.int32),
            pltpu.VMEM((NB, L, D), jnp.float32),
            pltpu.SemaphoreType.DMA((NB,)),
            pltpu.SemaphoreType.DMA((NB,)),
        ],
        compiler_params=pltpu.CompilerParams(use_tc_tiling_on_sc=False),
        interpret=interpret,
    )
    def emb_kernel(tok, tab, out, idx_v, ring, gsem, wsem):
        cc = lax.axis_index("c")
        ss = lax.axis_index("s")
        wid = ss * NC + cc
        base = wid * SPW

        pltpu.sync_copy(tok.at[pl.ds(base, SPW)], idx_v)

        def g_start(k, b):
            pltpu.async_copy(tab.at[idx_v.at[k]], ring.at[b], gsem.at[b])

        def g_wait(b):
            pltpu.make_async_copy(tab.at[pl.ds(0, L)], ring.at[b],
                                  gsem.at[b]).wait()

        def w_start(k, b):
            pltpu.async_copy(ring.at[b], out.at[base + k], wsem.at[b])

        def w_wait(k, b):
            pltpu.make_async_copy(ring.at[b], out.at[base + k],
                                  wsem.at[b]).wait()

        for b in range(NB):
            g_start(b, b)

        @pl.loop(0, SPW - NB, step=NB)
        def _(k0):
            for b in range(NB):
                k = k0 + b
                g_wait(b)
                w_start(k, b)
                w_wait(k, b)
                g_start(k + NB, b)

        for b in range(NB):
            g_wait(b)
            w_start(SPW - NB + b, b)
        for b in range(NB):
            w_wait(SPW - NB + b, b)

    return emb_kernel


_emb_t = _make_transposed_kernel()
_emb_bulk = _make_bulk_kernel()


def kernel(src_tokens, tgt_tokens, src_table, tgt_table):
    tgt_emb = _emb_bulk(tgt_tokens.astype(jnp.int32), tgt_table)
    y = _emb_t(jnp.transpose(src_tokens).astype(jnp.int32), src_table)
    src_emb = (y.reshape(L, DT, BT, DL, BL)
               .transpose(2, 4, 0, 1, 3).reshape(B, L, D))
    return src_emb, tgt_emb
